# baseline (speedup 1.0000x reference)
import functools

import jax
import jax.numpy as jnp
from jax import lax
from jax.experimental import pallas as pl
from jax.experimental.pallas import tpu as pltpu
from jax.experimental.pallas import tpu_sc as plsc

N = 20000
C = 81
P = 128

_NW = 32
_CHUNK = 624
_TAIL = N - (_NW - 1) * _CHUNK


@functools.partial(
    pl.kernel,
    out_type=jax.ShapeDtypeStruct((N,), jnp.float32),
    mesh=plsc.VectorSubcoreMesh(core_axis_name="c", subcore_axis_name="s"),
    scratch_types=[
        pltpu.VMEM((_TAIL,), jnp.int32),
        pltpu.VMEM((_TAIL,), jnp.float32),
        pltpu.SemaphoreType.DMA,
    ],
)
def _sc_gather_col(prob_flat_hbm, out_hbm, idx_ref, val_ref, sem):
    wid = lax.axis_index("s") * 2 + lax.axis_index("c")
    base = wid * _CHUNK
    iota = lax.iota(jnp.int32, 16)

    @pl.when(wid < _NW - 1)
    def _main():
        for j in range(_CHUNK // 16):
            idx_ref[pl.ds(16 * j, 16)] = (base + 16 * j + iota) * C
        pltpu.async_copy(
            prob_flat_hbm.at[idx_ref.at[pl.ds(0, _CHUNK)]],
            val_ref.at[pl.ds(0, _CHUNK)], sem).wait()
        pltpu.sync_copy(val_ref.at[pl.ds(0, _CHUNK)],
                        out_hbm.at[pl.ds(base, _CHUNK)])

    @pl.when(wid == _NW - 1)
    def _tail():
        for j in range(_TAIL // 16):
            idx_ref[pl.ds(16 * j, 16)] = (base + 16 * j + iota) * C
        pltpu.async_copy(prob_flat_hbm.at[idx_ref], val_ref, sem).wait()
        pltpu.sync_copy(val_ref, out_hbm.at[pl.ds(base, _TAIL)])


def _tc_body(col_ref, lab_ref, w_ref, pcl_ref, pcp_ref, imw_ref, iml_ref,
             out_ref):
    col = col_ref[...]
    lab = lab_ref[...]
    w = w_ref[...]
    bg_active = (iml_ref[0, 0] != 0.0).astype(jnp.float32)
    mask = (lab == 0).astype(jnp.float32)
    bg = -bg_active * jnp.sum(mask * w * jnp.log(col), axis=(0, 1),
                              keepdims=True)

    pcl = pcl_ref[...]
    iota_c = lax.broadcasted_iota(jnp.int32, (C, P), 0)
    onehot = (iota_c == pcl).astype(jnp.float32)
    gathered = lax.dot_general(
        iml_ref[...], onehot, (((1,), (0,)), ((), ())),
        preferred_element_type=jnp.float32)
    fg_active = (gathered != 0.0) & (pcl > 0)
    fg_vals = imw_ref[...] * jnp.log(pcp_ref[...])
    fg = -jnp.sum(jnp.where(fg_active, fg_vals, 0.0), axis=(0, 1),
                  keepdims=True)

    out_ref[...] = (bg + fg) * (1.0 / N)


@functools.partial(jax.jit, static_argnames=())
def kernel(pcl_prob, labels, cls_loss_weights, gt_assignment, pc_labels,
           pc_probs, pc_count, img_cls_loss_weights, im_labels_real):
    del gt_assignment, pc_count
    col = _sc_gather_col(pcl_prob.reshape(N * C))
    out = pl.pallas_call(
        _tc_body,
        out_shape=jax.ShapeDtypeStruct((1, 1), jnp.float32),
    )(col.reshape(1, N), labels.reshape(1, N),
      cls_loss_weights.reshape(1, N), pc_labels.reshape(1, P),
      pc_probs.reshape(1, P), img_cls_loss_weights.reshape(1, P),
      im_labels_real.reshape(1, C))
    return out[0, 0]


# speedup vs baseline: 3.3675x; 3.3675x over previous
import functools

import jax
import jax.numpy as jnp
from jax import lax
from jax.experimental import pallas as pl
from jax.experimental.pallas import tpu as pltpu

N = 20000
C = 81
P = 128
CHUNKS = 10
BLK = N // CHUNKS


def _tc_body(*refs):
    prob_any = refs[0]
    lab_refs = refs[1:1 + CHUNKS]
    w_refs = refs[1 + CHUNKS:1 + 2 * CHUNKS]
    pcl_ref, pcp_ref, imw_ref, iml_ref, out_ref = refs[1 + 2 * CHUNKS:
                                                       6 + 2 * CHUNKS]
    bufs = refs[6 + 2 * CHUNKS:6 + 3 * CHUNKS]
    sems = refs[6 + 3 * CHUNKS:6 + 4 * CHUNKS]

    def cp(k):
        return pltpu.make_async_copy(
            prob_any.at[pl.ds(BLK * k, BLK), :], bufs[k], sems[k])

    for k in range(CHUNKS):
        cp(k).start()

    e0 = (lax.broadcasted_iota(jnp.int32, (1, C), 1) == 0).astype(
        jnp.float32)
    bg_active = (iml_ref[0, 0] != 0.0).astype(jnp.float32)

    bg = jnp.zeros((1, 1), dtype=jnp.float32)
    for k in range(CHUNKS):
        cp(k).wait()
        col = lax.dot_general(
            e0, bufs[k][...], (((1,), (1,)), ((), ())),
            preferred_element_type=jnp.float32)
        mask = (lab_refs[k][0] == 0).astype(jnp.float32)
        bg = bg - jnp.sum(mask * w_refs[k][0] * jnp.log(col),
                          axis=(0, 1), keepdims=True)
    bg = bg * bg_active

    pcl = pcl_ref[...]
    iota_c = lax.broadcasted_iota(jnp.int32, (C, P), 0)
    onehot = (iota_c == pcl).astype(jnp.float32)
    gathered = lax.dot_general(
        iml_ref[...], onehot, (((1,), (0,)), ((), ())),
        preferred_element_type=jnp.float32)
    fg_active = (gathered != 0.0) & (pcl > 0)
    fg_vals = imw_ref[...] * jnp.log(pcp_ref[...])
    fg = -jnp.sum(jnp.where(fg_active, fg_vals, 0.0), axis=(0, 1),
                  keepdims=True)

    out_ref[...] = (bg + fg) * (1.0 / N)


@functools.partial(jax.jit, static_argnames=())
def kernel(pcl_prob, labels, cls_loss_weights, gt_assignment, pc_labels,
           pc_probs, pc_count, img_cls_loss_weights, im_labels_real):
    del gt_assignment, pc_count
    out = pl.pallas_call(
        _tc_body,
        grid=(1,),
        in_specs=(
            [pl.BlockSpec(memory_space=pltpu.MemorySpace.HBM)]
            + [pl.BlockSpec((1, 1, BLK), lambda i, k=k: (k, 0, 0))
               for k in range(CHUNKS)]
            + [pl.BlockSpec((1, 1, BLK), lambda i, k=k: (k, 0, 0))
               for k in range(CHUNKS)]
            + [
                pl.BlockSpec((1, P), lambda i: (0, 0)),
                pl.BlockSpec((1, P), lambda i: (0, 0)),
                pl.BlockSpec((1, P), lambda i: (0, 0)),
                pl.BlockSpec((1, C), lambda i: (0, 0)),
            ]
        ),
        out_specs=pl.BlockSpec((1, 1), lambda i: (0, 0)),
        out_shape=jax.ShapeDtypeStruct((1, 1), jnp.float32),
        scratch_shapes=(
            [pltpu.VMEM((BLK, C), jnp.float32) for _ in range(CHUNKS)]
            + [pltpu.SemaphoreType.DMA for _ in range(CHUNKS)]
        ),
    )(pcl_prob,
      *([labels.reshape(CHUNKS, 1, BLK)] * CHUNKS),
      *([cls_loss_weights.reshape(CHUNKS, 1, BLK)] * CHUNKS),
      pc_labels.reshape(1, P), pc_probs.reshape(1, P),
      img_cls_loss_weights.reshape(1, P), im_labels_real.reshape(1, C))
    return out[0, 0]


# speedup vs baseline: 33.0967x; 9.8282x over previous
import functools

import jax
import jax.numpy as jnp
from jax import lax
from jax.experimental import pallas as pl
from jax.experimental.pallas import tpu as pltpu

N = 20000
C = 81
P = 128


def _tc_body(colt_ref, lab_ref, w_ref, pcl_ref, pcp_ref, imw_ref, iml_ref,
             out_ref):
    col = colt_ref[0:1, :]
    lab = lab_ref[...].reshape(1, N)
    w = w_ref[...].reshape(1, N)
    bg_active = (iml_ref[0, 0] != 0.0).astype(jnp.float32)
    mask = (lab == 0).astype(jnp.float32)
    bg = -bg_active * jnp.sum(mask * w * jnp.log(col), axis=(0, 1),
                              keepdims=True)

    pcl = pcl_ref[...]
    iota_c = lax.broadcasted_iota(jnp.int32, (C, P), 0)
    onehot = (iota_c == pcl).astype(jnp.float32)
    gathered = lax.dot_general(
        iml_ref[...], onehot, (((1,), (0,)), ((), ())),
        preferred_element_type=jnp.float32)
    fg_active = (gathered != 0.0) & (pcl > 0)
    fg_vals = imw_ref[...] * jnp.log(pcp_ref[...])
    fg = -jnp.sum(jnp.where(fg_active, fg_vals, 0.0), axis=(0, 1),
                  keepdims=True)

    out_ref[...] = (bg + fg) * (1.0 / N)


@functools.partial(jax.jit, static_argnames=())
def kernel(pcl_prob, labels, cls_loss_weights, gt_assignment, pc_labels,
           pc_probs, pc_count, img_cls_loss_weights, im_labels_real):
    del gt_assignment, pc_count
    probt = pcl_prob.T
    out = pl.pallas_call(
        _tc_body,
        grid=(1,),
        in_specs=[
            pl.BlockSpec((8, N), lambda i: (0, 0)),
            pl.BlockSpec((N,), lambda i: (0,)),
            pl.BlockSpec((N,), lambda i: (0,)),
            pl.BlockSpec((1, P), lambda i: (0, 0)),
            pl.BlockSpec((1, P), lambda i: (0, 0)),
            pl.BlockSpec((1, P), lambda i: (0, 0)),
            pl.BlockSpec((1, C), lambda i: (0, 0)),
        ],
        out_specs=pl.BlockSpec((1, 1), lambda i: (0, 0)),
        out_shape=jax.ShapeDtypeStruct((1, 1), jnp.float32),
    )(probt, labels, cls_loss_weights,
      pc_labels.reshape(1, P), pc_probs.reshape(1, P),
      img_cls_loss_weights.reshape(1, P), im_labels_real.reshape(1, C))
    return out[0, 0]
